# baseline (speedup 1.0000x reference)
import dataclasses
import functools

import jax
import jax.numpy as jnp
from jax import lax
from jax.experimental import pallas as pl
from jax.experimental.pallas import tpu as pltpu
from jax.experimental.pallas import tpu_sc as plsc

_N = 10000
_D = 128
_H = 128
_OUT = 64
_E = 300000
_EROWS = 2400
_EPAD = _EROWS * 128
_ACCR = 10240
_ZR = _ACCR // 16
_NB2 = _EROWS // 32
_EW = _EPAD // 16
_GR = 128
_SC = 40
_C1, _R1 = 5, 30
_C2, _R2 = 2, 38
_EROWS2 = 32 * _C2 * _R2

_f32 = jnp.float32
_SDS = jax.ShapeDtypeStruct


@functools.lru_cache(maxsize=None)
def _vmesh():
    return plsc.VectorSubcoreMesh(core_axis_name="c", subcore_axis_name="s")


@functools.lru_cache(maxsize=None)
def _sc_params():
    cp = pltpu.CompilerParams()
    if "needs_layout_passes" in pltpu.CompilerParams.__dataclass_fields__:
        cp = dataclasses.replace(cp, needs_layout_passes=False)
    return cp


def _build_deg_kernel():
    @functools.partial(
        pl.kernel,
        out_type=_SDS((2, _ACCR), _f32),
        mesh=_vmesh(),
        compiler_params=_sc_params(),
        scratch_types=[
            pltpu.VMEM((_EW,), jnp.int32),
            pltpu.VMEM((_ACCR,), _f32),
            pltpu.VMEM((_ZR,), _f32),
            pltpu.VMEM((_ZR,), _f32),
            pltpu.VMEM_SHARED((16, _ACCR), _f32),
        ],
    )
    def deg_kernel(d_all, inv_all, dstv, hist, rbuf, tbuf, shr):
        c = lax.axis_index("c")
        s = lax.axis_index("s")
        zeros16 = jnp.zeros((16,), _f32)
        ones16 = jnp.ones((16,), _f32)

        @pl.loop(0, _ACCR, step=16)
        def _(i):
            hist[pl.ds(i, 16)] = zeros16

        pltpu.sync_copy(d_all.at[c, s], dstv)

        @pl.loop(0, _EW, step=16)
        def _(k):
            idx = dstv[pl.ds(k, 16)]
            plsc.addupdate_scatter(hist, [idx], ones16)

        pltpu.sync_copy(hist, shr.at[s])
        plsc.subcore_barrier()
        pltpu.sync_copy(shr.at[0, pl.ds(s * _ZR, _ZR)], rbuf)
        for t in range(1, 16):
            pltpu.sync_copy(shr.at[t, pl.ds(s * _ZR, _ZR)], tbuf)

            @pl.loop(0, _ZR, step=16)
            def _(i):
                rbuf[pl.ds(i, 16)] = rbuf[pl.ds(i, 16)] + tbuf[pl.ds(i, 16)]

        @pl.loop(0, _ZR, step=16)
        def _(i):
            d = rbuf[pl.ds(i, 16)]
            rbuf[pl.ds(i, 16)] = 1.0 / jnp.maximum(d, 1.0)

        pltpu.sync_copy(rbuf, inv_all.at[c, pl.ds(s * _ZR, _ZR)])

    return deg_kernel


def _edge_pass(table, srcv, dstv, bufs, semsG, semsS, acc, crows):
    ring = ((bufs[0], semsG[0]), (bufs[1], semsG[1]))
    pltpu.async_copy(table.at[srcv.at[0]], ring[0][0], ring[0][1])
    pltpu.async_copy(table.at[srcv.at[1]], ring[1][0], ring[1][1])

    @pl.loop(0, crows - 2, step=2)
    def _(t):
        for b, (buf, sem) in enumerate(ring):
            j = t + b
            pltpu.make_async_copy(table.at[srcv.at[j]], buf, sem).wait()
            pltpu.sync_copy(buf, acc.at[dstv.at[j]], add=True)
            pltpu.async_copy(table.at[srcv.at[j + 2]], buf, sem)

    for b, (buf, sem) in enumerate(ring):
        j = crows - 2 + b
        pltpu.make_async_copy(table.at[srcv.at[j]], buf, sem).wait()
        pltpu.sync_copy(buf, acc.at[dstv.at[j]], add=True)


def _scale_out(acc, invv, buf, s, write_chunk):
    for chunk in range(_ZR // _SC):
        base = s * _ZR + chunk * _SC
        bv = buf.at[pl.ds(0, _SC)]
        pltpu.sync_copy(acc.at[pl.ds(base, _SC)], bv)

        lanes0 = lax.iota(jnp.int32, 16)

        @pl.loop(0, _SC)
        def _(r):
            rr = jnp.full((16,), chunk * _SC + r, jnp.int32)
            vv = plsc.load_gather(invv, [rr])
            rI = jnp.full((16,), r, jnp.int32)
            for l in range(8):
                lanes = lanes0 + (l * 16)
                v = plsc.load_gather(buf, [rI, lanes])
                plsc.store_scatter(buf, [rI, lanes], v * vv)

        write_chunk(base, bv)


def _build_agg1_kernel():
    @functools.partial(
        pl.kernel,
        out_type=(_SDS((_ACCR, _H), _f32), _SDS((_ACCR, _H), _f32)),
        mesh=_vmesh(),
        compiler_params=_sc_params(),
        scratch_types=[
            pltpu.VMEM((_R1, _GR), jnp.int32),
            pltpu.VMEM((_R1, _GR), jnp.int32),
            pltpu.VMEM((_GR, _H), _f32),
            pltpu.VMEM((_GR, _H), _f32),
            pltpu.VMEM((_ZR,), _f32),
            pltpu.VMEM_SHARED((_ACCR, _H), _f32),
        ] + [pltpu.SemaphoreType.DMA] * 2,
    )
    def agg1_kernel(ha, hb, sab, dab, sba, dba, z, inv_all, meanb, meana,
                    srcv, dstv, b0, b1, invv, acc, *sems):
        c = lax.axis_index("c")
        s = lax.axis_index("s")
        bufs, semsG, semsS = (b0, b1), sems, sems
        pltpu.sync_copy(z, acc.at[pl.ds(s * _ZR, _ZR)])
        plsc.subcore_barrier()

        def run(table, s4d, d4d, ci, out):
            for ch in range(_C1):
                pltpu.sync_copy(s4d.at[s, ch], srcv)
                pltpu.sync_copy(d4d.at[s, ch], dstv)
                _edge_pass(table, srcv, dstv, bufs, semsG, semsS, acc, _R1)

            plsc.subcore_barrier()
            pltpu.sync_copy(inv_all.at[ci, pl.ds(s * _ZR, _ZR)], invv)
            _scale_out(acc, invv, b0, s,
                       lambda base, b: pltpu.sync_copy(b, out.at[pl.ds(base, _SC)]))

        @pl.when(c == 0)
        def _():
            run(ha, sab, dab, 0, meanb)

        @pl.when(c == 1)
        def _():
            run(hb, sba, dba, 1, meana)

    return agg1_kernel


def _build_agg2_kernel():
    @functools.partial(
        pl.kernel,
        out_type=_SDS((2, _ACCR, _H), _f32),
        mesh=_vmesh(),
        compiler_params=_sc_params(),
        scratch_types=[
            pltpu.VMEM((_R2, _GR), jnp.int32),
            pltpu.VMEM((_R2, _GR), jnp.int32),
            pltpu.VMEM((_GR, _H), _f32),
            pltpu.VMEM((_GR, _H), _f32),
            pltpu.VMEM((_ZR,), _f32),
            pltpu.VMEM_SHARED((_ACCR, _H), _f32),
        ] + [pltpu.SemaphoreType.DMA] * 2,
    )
    def agg2_kernel(hb1, sba, dba, z, inv_all, p_all,
                    srcv, dstv, b0, b1, invv, acc, *sems):
        c = lax.axis_index("c")
        s = lax.axis_index("s")
        bufs, semsG, semsS = (b0, b1), sems, sems
        pltpu.sync_copy(z, acc.at[pl.ds(s * _ZR, _ZR)])
        plsc.subcore_barrier()

        w = c * 16 + s
        for ch in range(_C2):
            pltpu.sync_copy(sba.at[w, ch], srcv)
            pltpu.sync_copy(dba.at[w, ch], dstv)
            _edge_pass(hb1, srcv, dstv, bufs, semsG, semsS, acc, _R2)

        plsc.subcore_barrier()
        pltpu.sync_copy(inv_all.at[1, pl.ds(s * _ZR, _ZR)], invv)
        _scale_out(acc, invv, b0, s,
                   lambda base, b: pltpu.sync_copy(b, p_all.at[c, pl.ds(base, _SC)]))

    return agg2_kernel


_deg_kernel = None
_agg1_kernel = None
_agg2_kernel = None


def _sc_kernels():
    global _deg_kernel, _agg1_kernel, _agg2_kernel
    if _deg_kernel is None:
        _deg_kernel = _build_deg_kernel()
        _agg1_kernel = _build_agg1_kernel()
        _agg2_kernel = _build_agg2_kernel()
    return _deg_kernel, _agg1_kernel, _agg2_kernel


_R = 1000


def _mm_bias_relu(x, w, b2d):
    def body(x_ref, w_ref, b_ref, o_ref):
        o_ref[...] = jnp.maximum(
            jnp.dot(x_ref[...], w_ref[...], preferred_element_type=_f32)
            + b_ref[...], 0.0)

    h = w.shape[1]
    return pl.pallas_call(
        body,
        grid=(_N // _R,),
        in_specs=[
            pl.BlockSpec((_R, x.shape[1]), lambda i: (i, 0)),
            pl.BlockSpec((x.shape[1], h), lambda i: (0, 0)),
            pl.BlockSpec((1, h), lambda i: (0, 0)),
        ],
        out_specs=pl.BlockSpec((_R, h), lambda i: (i, 0)),
        out_shape=_SDS((_N, h), _f32),
    )(x, w, b2d)


def _combine(mean, h, wl, bl2d, wr):
    def body(m_ref, h_ref, wl_ref, bl_ref, wr_ref, o_ref):
        o_ref[...] = jnp.maximum(
            jnp.dot(m_ref[...], wl_ref[...], preferred_element_type=_f32)
            + jnp.dot(h_ref[...], wr_ref[...], preferred_element_type=_f32)
            + bl_ref[...], 0.0)

    return pl.pallas_call(
        body,
        grid=(_N // _R,),
        in_specs=[
            pl.BlockSpec((_R, _H), lambda i: (i, 0)),
            pl.BlockSpec((_R, _H), lambda i: (i, 0)),
            pl.BlockSpec((_H, _H), lambda i: (0, 0)),
            pl.BlockSpec((1, _H), lambda i: (0, 0)),
            pl.BlockSpec((_H, _H), lambda i: (0, 0)),
        ],
        out_specs=pl.BlockSpec((_R, _H), lambda i: (i, 0)),
        out_shape=_SDS((_N, _H), _f32),
    )(mean, h, wl, bl2d, wr)


def _combine2_out(p_all, h, wl, bl2d, wr, wo, bo2d):
    def body(p_ref, h_ref, wl_ref, bl_ref, wr_ref, wo_ref, bo_ref, o_ref):
        agg = p_ref[0] + p_ref[1]
        t = jnp.maximum(
            jnp.dot(agg, wl_ref[...], preferred_element_type=_f32)
            + jnp.dot(h_ref[...], wr_ref[...], preferred_element_type=_f32)
            + bl_ref[...], 0.0)
        o_ref[...] = (jnp.dot(t, wo_ref[...], preferred_element_type=_f32)
                      + bo_ref[...])

    return pl.pallas_call(
        body,
        grid=(_N // _R,),
        in_specs=[
            pl.BlockSpec((2, _R, _H), lambda i: (0, i, 0)),
            pl.BlockSpec((_R, _H), lambda i: (i, 0)),
            pl.BlockSpec((_H, _H), lambda i: (0, 0)),
            pl.BlockSpec((1, _H), lambda i: (0, 0)),
            pl.BlockSpec((_H, _H), lambda i: (0, 0)),
            pl.BlockSpec((_H, _OUT), lambda i: (0, 0)),
            pl.BlockSpec((1, _OUT), lambda i: (0, 0)),
        ],
        out_specs=pl.BlockSpec((_R, _OUT), lambda i: (i, 0)),
        out_shape=_SDS((_N, _OUT), _f32),
    )(p_all, h, wl, bl2d, wr, wo, bo2d)


def kernel(x_type_a, x_type_b, edge_index_ab, edge_index_ba,
           W_in_a, b_in_a, W_in_b, b_in_b,
           Wl1_ab, bl1_ab, Wr1_ab, Wl1_ba, bl1_ba, Wr1_ba,
           Wl2_ab, bl2_ab, Wr2_ab, Wl2_ba, bl2_ba, Wr2_ba,
           W_out, b_out):
    del Wl2_ab, bl2_ab, Wr2_ab
    deg_kernel, agg1_kernel, agg2_kernel = _sc_kernels()

    i32 = jnp.int32

    def prep(ei, shp):
        npad = 1
        for d in shp:
            npad *= d
        npad -= _E
        pad_src = (jnp.arange(npad, dtype=i32) * 97) % _N
        pad_dst = _N + (jnp.arange(npad, dtype=i32) % (_ACCR - _N))
        src = jnp.concatenate([ei[0], pad_src])
        dst = jnp.concatenate([ei[1], pad_dst])
        return src.reshape(shp), dst.reshape(shp)

    shp16 = (16, _C1, _R1, _GR)
    sab, dab = prep(edge_index_ab, shp16)
    sba, dba = prep(edge_index_ba, shp16)
    sba32, dba32 = prep(edge_index_ba, (32, _C2, _R2, _GR))
    d_all = jnp.stack([dab, dba]).reshape(2, 16, _EW)
    z = jnp.zeros((_ZR, _H), _f32)

    inv_all = deg_kernel(d_all)

    ha0 = _mm_bias_relu(x_type_a, W_in_a, b_in_a.reshape(1, _H))
    hb0 = _mm_bias_relu(x_type_b, W_in_b, b_in_b.reshape(1, _H))

    meanb, meana = agg1_kernel(ha0, hb0, sab, dab, sba, dba, z, inv_all)

    hb1 = _combine(meanb, hb0, Wl1_ab, bl1_ab.reshape(1, _H), Wr1_ab)
    ha1 = _combine(meana, ha0, Wl1_ba, bl1_ba.reshape(1, _H), Wr1_ba)

    p_all = agg2_kernel(hb1, sba32, dba32, z, inv_all)

    return _combine2_out(p_all, ha1,
                         Wl2_ba, bl2_ba.reshape(1, _H), Wr2_ba,
                         W_out, b_out.reshape(1, _OUT))


# speedup vs baseline: 1.0302x; 1.0302x over previous
import dataclasses
import functools

import jax
import jax.numpy as jnp
from jax import lax
from jax.experimental import pallas as pl
from jax.experimental.pallas import tpu as pltpu
from jax.experimental.pallas import tpu_sc as plsc

_N = 10000
_D = 128
_H = 128
_OUT = 64
_E = 300000
_EROWS = 2400
_EPAD = _EROWS * 128
_ACCR = 10240
_ZR = _ACCR // 16
_NB2 = _EROWS // 32
_EW = _EPAD // 16
_GR = 128
_SC = 128
_C1, _R1 = 3, 50
_C2, _R2 = 2, 38
_EROWS2 = 32 * _C2 * _R2

_f32 = jnp.float32
_SDS = jax.ShapeDtypeStruct


@functools.lru_cache(maxsize=None)
def _vmesh():
    return plsc.VectorSubcoreMesh(core_axis_name="c", subcore_axis_name="s")


@functools.lru_cache(maxsize=None)
def _sc_params():
    cp = pltpu.CompilerParams()
    if "needs_layout_passes" in pltpu.CompilerParams.__dataclass_fields__:
        cp = dataclasses.replace(cp, needs_layout_passes=False)
    return cp


def _build_deg_kernel():
    @functools.partial(
        pl.kernel,
        out_type=_SDS((2, _ACCR), _f32),
        mesh=_vmesh(),
        compiler_params=_sc_params(),
        scratch_types=[
            pltpu.VMEM((_EW,), jnp.int32),
            pltpu.VMEM((_ACCR,), _f32),
            pltpu.VMEM((_ZR,), _f32),
            pltpu.VMEM((_ZR,), _f32),
            pltpu.VMEM_SHARED((16, _ACCR), _f32),
        ],
    )
    def deg_kernel(d_all, inv_all, dstv, hist, rbuf, tbuf, shr):
        c = lax.axis_index("c")
        s = lax.axis_index("s")
        zeros16 = jnp.zeros((16,), _f32)
        ones16 = jnp.ones((16,), _f32)

        @pl.loop(0, _ACCR, step=16)
        def _(i):
            hist[pl.ds(i, 16)] = zeros16

        pltpu.sync_copy(d_all.at[c, s], dstv)

        @pl.loop(0, _EW, step=16)
        def _(k):
            idx = dstv[pl.ds(k, 16)]
            plsc.addupdate_scatter(hist, [idx], ones16)

        pltpu.sync_copy(hist, shr.at[s])
        plsc.subcore_barrier()
        pltpu.sync_copy(shr.at[0, pl.ds(s * _ZR, _ZR)], rbuf)
        for t in range(1, 16):
            pltpu.sync_copy(shr.at[t, pl.ds(s * _ZR, _ZR)], tbuf)

            @pl.loop(0, _ZR, step=16)
            def _(i):
                rbuf[pl.ds(i, 16)] = rbuf[pl.ds(i, 16)] + tbuf[pl.ds(i, 16)]

        @pl.loop(0, _ZR, step=16)
        def _(i):
            d = rbuf[pl.ds(i, 16)]
            rbuf[pl.ds(i, 16)] = 1.0 / jnp.maximum(d, 1.0)

        pltpu.sync_copy(rbuf, inv_all.at[c, pl.ds(s * _ZR, _ZR)])

    return deg_kernel


def _edge_pass(table, srcv, dstv, bufs, semsG, semsS, acc, crows):
    ring = ((bufs[0], semsG[0]), (bufs[1], semsG[1]))
    pltpu.async_copy(table.at[srcv.at[0]], ring[0][0], ring[0][1])
    pltpu.async_copy(table.at[srcv.at[1]], ring[1][0], ring[1][1])

    @pl.loop(0, crows - 2, step=2)
    def _(t):
        for b, (buf, sem) in enumerate(ring):
            j = t + b
            pltpu.make_async_copy(table.at[srcv.at[j]], buf, sem).wait()
            pltpu.sync_copy(buf, acc.at[dstv.at[j]], add=True)
            pltpu.async_copy(table.at[srcv.at[j + 2]], buf, sem)

    for b, (buf, sem) in enumerate(ring):
        j = crows - 2 + b
        pltpu.make_async_copy(table.at[srcv.at[j]], buf, sem).wait()
        pltpu.sync_copy(buf, acc.at[dstv.at[j]], add=True)


def _scale_out(acc, invv, buf, s, write_chunk):
    for chunk in range(_ZR // _SC):
        base = s * _ZR + chunk * _SC
        bv = buf.at[pl.ds(0, _SC)]
        pltpu.sync_copy(acc.at[pl.ds(base, _SC)], bv)

        lanes0 = lax.iota(jnp.int32, 16)

        @pl.loop(0, _SC)
        def _(r):
            rr = jnp.full((16,), chunk * _SC + r, jnp.int32)
            vv = plsc.load_gather(invv, [rr])
            rI = jnp.full((16,), r, jnp.int32)
            for l in range(8):
                lanes = lanes0 + (l * 16)
                v = plsc.load_gather(buf, [rI, lanes])
                plsc.store_scatter(buf, [rI, lanes], v * vv)

        write_chunk(base, bv)


def _build_agg1_kernel():
    @functools.partial(
        pl.kernel,
        out_type=(_SDS((_ACCR, _H), _f32), _SDS((_ACCR, _H), _f32)),
        mesh=_vmesh(),
        compiler_params=_sc_params(),
        scratch_types=[
            pltpu.VMEM((_R1, _GR), jnp.int32),
            pltpu.VMEM((_R1, _GR), jnp.int32),
            pltpu.VMEM((_GR, _H), _f32),
            pltpu.VMEM((_GR, _H), _f32),
            pltpu.VMEM((_ZR,), _f32),
            pltpu.VMEM_SHARED((_ACCR, _H), _f32),
        ] + [pltpu.SemaphoreType.DMA] * 2,
    )
    def agg1_kernel(ha, hb, sab, dab, sba, dba, z, inv_all, meanb, meana,
                    srcv, dstv, b0, b1, invv, acc, *sems):
        c = lax.axis_index("c")
        s = lax.axis_index("s")
        bufs, semsG, semsS = (b0, b1), sems, sems
        pltpu.sync_copy(z, acc.at[pl.ds(s * _ZR, _ZR)])
        plsc.subcore_barrier()

        def run(table, s4d, d4d, ci, out):
            for ch in range(_C1):
                pltpu.sync_copy(s4d.at[s, ch], srcv)
                pltpu.sync_copy(d4d.at[s, ch], dstv)
                _edge_pass(table, srcv, dstv, bufs, semsG, semsS, acc, _R1)

            plsc.subcore_barrier()
            pltpu.sync_copy(inv_all.at[ci, pl.ds(s * _ZR, _ZR)], invv)
            _scale_out(acc, invv, b0, s,
                       lambda base, b: pltpu.sync_copy(b, out.at[pl.ds(base, _SC)]))

        @pl.when(c == 0)
        def _():
            run(ha, sab, dab, 0, meanb)

        @pl.when(c == 1)
        def _():
            run(hb, sba, dba, 1, meana)

    return agg1_kernel


def _build_agg2_kernel():
    @functools.partial(
        pl.kernel,
        out_type=_SDS((2, _ACCR, _H), _f32),
        mesh=_vmesh(),
        compiler_params=_sc_params(),
        scratch_types=[
            pltpu.VMEM((_R2, _GR), jnp.int32),
            pltpu.VMEM((_R2, _GR), jnp.int32),
            pltpu.VMEM((_GR, _H), _f32),
            pltpu.VMEM((_GR, _H), _f32),
            pltpu.VMEM((_ZR,), _f32),
            pltpu.VMEM_SHARED((_ACCR, _H), _f32),
        ] + [pltpu.SemaphoreType.DMA] * 2,
    )
    def agg2_kernel(hb1, sba, dba, z, inv_all, p_all,
                    srcv, dstv, b0, b1, invv, acc, *sems):
        c = lax.axis_index("c")
        s = lax.axis_index("s")
        bufs, semsG, semsS = (b0, b1), sems, sems
        pltpu.sync_copy(z, acc.at[pl.ds(s * _ZR, _ZR)])
        plsc.subcore_barrier()

        w = c * 16 + s
        for ch in range(_C2):
            pltpu.sync_copy(sba.at[w, ch], srcv)
            pltpu.sync_copy(dba.at[w, ch], dstv)
            _edge_pass(hb1, srcv, dstv, bufs, semsG, semsS, acc, _R2)

        plsc.subcore_barrier()
        pltpu.sync_copy(inv_all.at[1, pl.ds(s * _ZR, _ZR)], invv)
        _scale_out(acc, invv, b0, s,
                   lambda base, b: pltpu.sync_copy(b, p_all.at[c, pl.ds(base, _SC)]))

    return agg2_kernel


_deg_kernel = None
_agg1_kernel = None
_agg2_kernel = None


def _sc_kernels():
    global _deg_kernel, _agg1_kernel, _agg2_kernel
    if _deg_kernel is None:
        _deg_kernel = _build_deg_kernel()
        _agg1_kernel = _build_agg1_kernel()
        _agg2_kernel = _build_agg2_kernel()
    return _deg_kernel, _agg1_kernel, _agg2_kernel


_R = 1000


def _mm_bias_relu(x, w, b2d):
    def body(x_ref, w_ref, b_ref, o_ref):
        o_ref[...] = jnp.maximum(
            jnp.dot(x_ref[...], w_ref[...], preferred_element_type=_f32)
            + b_ref[...], 0.0)

    h = w.shape[1]
    return pl.pallas_call(
        body,
        grid=(_N // _R,),
        in_specs=[
            pl.BlockSpec((_R, x.shape[1]), lambda i: (i, 0)),
            pl.BlockSpec((x.shape[1], h), lambda i: (0, 0)),
            pl.BlockSpec((1, h), lambda i: (0, 0)),
        ],
        out_specs=pl.BlockSpec((_R, h), lambda i: (i, 0)),
        out_shape=_SDS((_N, h), _f32),
    )(x, w, b2d)


def _combine(mean, h, wl, bl2d, wr):
    def body(m_ref, h_ref, wl_ref, bl_ref, wr_ref, o_ref):
        o_ref[...] = jnp.maximum(
            jnp.dot(m_ref[...], wl_ref[...], preferred_element_type=_f32)
            + jnp.dot(h_ref[...], wr_ref[...], preferred_element_type=_f32)
            + bl_ref[...], 0.0)

    return pl.pallas_call(
        body,
        grid=(_N // _R,),
        in_specs=[
            pl.BlockSpec((_R, _H), lambda i: (i, 0)),
            pl.BlockSpec((_R, _H), lambda i: (i, 0)),
            pl.BlockSpec((_H, _H), lambda i: (0, 0)),
            pl.BlockSpec((1, _H), lambda i: (0, 0)),
            pl.BlockSpec((_H, _H), lambda i: (0, 0)),
        ],
        out_specs=pl.BlockSpec((_R, _H), lambda i: (i, 0)),
        out_shape=_SDS((_N, _H), _f32),
    )(mean, h, wl, bl2d, wr)


def _combine2_out(p_all, h, wl, bl2d, wr, wo, bo2d):
    def body(p_ref, h_ref, wl_ref, bl_ref, wr_ref, wo_ref, bo_ref, o_ref):
        agg = p_ref[0] + p_ref[1]
        t = jnp.maximum(
            jnp.dot(agg, wl_ref[...], preferred_element_type=_f32)
            + jnp.dot(h_ref[...], wr_ref[...], preferred_element_type=_f32)
            + bl_ref[...], 0.0)
        o_ref[...] = (jnp.dot(t, wo_ref[...], preferred_element_type=_f32)
                      + bo_ref[...])

    return pl.pallas_call(
        body,
        grid=(_N // _R,),
        in_specs=[
            pl.BlockSpec((2, _R, _H), lambda i: (0, i, 0)),
            pl.BlockSpec((_R, _H), lambda i: (i, 0)),
            pl.BlockSpec((_H, _H), lambda i: (0, 0)),
            pl.BlockSpec((1, _H), lambda i: (0, 0)),
            pl.BlockSpec((_H, _H), lambda i: (0, 0)),
            pl.BlockSpec((_H, _OUT), lambda i: (0, 0)),
            pl.BlockSpec((1, _OUT), lambda i: (0, 0)),
        ],
        out_specs=pl.BlockSpec((_R, _OUT), lambda i: (i, 0)),
        out_shape=_SDS((_N, _OUT), _f32),
    )(p_all, h, wl, bl2d, wr, wo, bo2d)


def kernel(x_type_a, x_type_b, edge_index_ab, edge_index_ba,
           W_in_a, b_in_a, W_in_b, b_in_b,
           Wl1_ab, bl1_ab, Wr1_ab, Wl1_ba, bl1_ba, Wr1_ba,
           Wl2_ab, bl2_ab, Wr2_ab, Wl2_ba, bl2_ba, Wr2_ba,
           W_out, b_out):
    del Wl2_ab, bl2_ab, Wr2_ab
    deg_kernel, agg1_kernel, agg2_kernel = _sc_kernels()

    i32 = jnp.int32

    def prep(ei, shp):
        npad = 1
        for d in shp:
            npad *= d
        npad -= _E
        pad_src = (jnp.arange(npad, dtype=i32) * 97) % _N
        pad_dst = _N + (jnp.arange(npad, dtype=i32) % (_ACCR - _N))
        src = jnp.concatenate([ei[0], pad_src])
        dst = jnp.concatenate([ei[1], pad_dst])
        return src.reshape(shp), dst.reshape(shp)

    shp16 = (16, _C1, _R1, _GR)
    sab, dab = prep(edge_index_ab, shp16)
    sba, dba = prep(edge_index_ba, shp16)
    sba32, dba32 = prep(edge_index_ba, (32, _C2, _R2, _GR))
    d_all = jnp.stack([dab, dba]).reshape(2, 16, _EW)
    z = jnp.zeros((_ZR, _H), _f32)

    inv_all = deg_kernel(d_all)

    ha0 = _mm_bias_relu(x_type_a, W_in_a, b_in_a.reshape(1, _H))
    hb0 = _mm_bias_relu(x_type_b, W_in_b, b_in_b.reshape(1, _H))

    meanb, meana = agg1_kernel(ha0, hb0, sab, dab, sba, dba, z, inv_all)

    hb1 = _combine(meanb, hb0, Wl1_ab, bl1_ab.reshape(1, _H), Wr1_ab)
    ha1 = _combine(meana, ha0, Wl1_ba, bl1_ba.reshape(1, _H), Wr1_ba)

    p_all = agg2_kernel(hb1, sba32, dba32, z, inv_all)

    return _combine2_out(p_all, ha1,
                         Wl2_ba, bl2_ba.reshape(1, _H), Wr2_ba,
                         W_out, b_out.reshape(1, _OUT))
